# baseline (speedup 1.0000x reference)
import functools

import jax
import jax.numpy as jnp
from jax import lax
from jax.experimental import pallas as pl
from jax.experimental.pallas import tpu as pltpu
from jax.experimental.pallas import tpu_sc as plsc

SOM0, SOM1 = 64, 128
K = SOM0 * SOM1
LATENT = 256
IN_DIM = 1024
B = 4096

BB = 128
NB = B // BB
KPAD = 8
ZROW = K

NC, NS = 2, 16
NW = NC * NS
BPW = B // NW
CH = 32
NCH = BPW // CH
LV = LATENT // 16

XB = 512
NXB = B // XB


def _tc_body(x_ref, we_ref, be_ref, emb_ref, wde_ref, bde_ref,
             ze_ref, dist_ref, k_ref, xe_ref,
             e2_ref, semb_ref, ids_ref):
    i = pl.program_id(0)

    @pl.when(i == 0)
    def _init():
        emb = emb_ref[...]
        e2_ref[...] = jnp.sum(emb * emb, axis=1)[None, :]
        semb_ref[...] = -2.0 * emb
        ids_ref[...] = lax.broadcasted_iota(jnp.int32, (1, K), 1)

    z = jnp.dot(x_ref[...], we_ref[...],
                preferred_element_type=jnp.float32) + be_ref[...]
    ze_ref[...] = z
    z2 = jnp.sum(z * z, axis=1, keepdims=True)
    cross2 = lax.dot_general(z, semb_ref[...], (((1,), (1,)), ((), ())),
                             preferred_element_type=jnp.float32)
    dist = (z2 + cross2) + e2_ref[...]
    dist_ref[...] = dist
    m = jnp.min(dist, axis=1, keepdims=True)
    k_ref[0, 0, :] = jnp.min(jnp.where(dist == m, ids_ref[...], K), axis=1)
    xe_ref[...] = jnp.dot(z, wde_ref[...],
                          preferred_element_type=jnp.float32) + bde_ref[...]


_tc_call = pl.pallas_call(
    _tc_body,
    grid=(NB,),
    in_specs=[
        pl.BlockSpec((BB, IN_DIM), lambda i: (i, 0)),
        pl.BlockSpec((IN_DIM, LATENT), lambda i: (0, 0)),
        pl.BlockSpec((1, LATENT), lambda i: (0, 0)),
        pl.BlockSpec((K, LATENT), lambda i: (0, 0)),
        pl.BlockSpec((LATENT, IN_DIM), lambda i: (0, 0)),
        pl.BlockSpec((1, IN_DIM), lambda i: (0, 0)),
    ],
    out_specs=[
        pl.BlockSpec((BB, LATENT), lambda i: (i, 0)),
        pl.BlockSpec((BB, K), lambda i: (i, 0)),
        pl.BlockSpec((1, 1, BB), lambda i: (i, 0, 0)),
        pl.BlockSpec((BB, IN_DIM), lambda i: (i, 0)),
    ],
    out_shape=[
        jax.ShapeDtypeStruct((B, LATENT), jnp.float32),
        jax.ShapeDtypeStruct((B, K), jnp.float32),
        jax.ShapeDtypeStruct((NB, 1, BB), jnp.int32),
        jax.ShapeDtypeStruct((B, IN_DIM), jnp.float32),
    ],
    scratch_shapes=[
        pltpu.VMEM((1, K), jnp.float32),
        pltpu.VMEM((K, LATENT), jnp.float32),
        pltpu.VMEM((1, K), jnp.int32),
    ],
)


def _xq_body(zq_ref, zup_ref, zdn_ref, zlf_ref, wdq_ref, bdq_ref,
             xq_ref, zn_ref):
    zq = zq_ref[...]
    xq_ref[...] = jnp.dot(zq, wdq_ref[...],
                          preferred_element_type=jnp.float32) + bdq_ref[...]
    zn_ref[:, 0, :] = zq
    zn_ref[:, 1, :] = zup_ref[...]
    zn_ref[:, 2, :] = zdn_ref[...]
    zn_ref[:, 3, :] = jnp.zeros((XB, LATENT), jnp.float32)
    zn_ref[:, 4, :] = zlf_ref[...]


_xq_call = pl.pallas_call(
    _xq_body,
    grid=(NXB,),
    in_specs=[
        pl.BlockSpec((XB, LATENT), lambda i: (i, 0)),
        pl.BlockSpec((XB, LATENT), lambda i: (i, 0)),
        pl.BlockSpec((XB, LATENT), lambda i: (i, 0)),
        pl.BlockSpec((XB, LATENT), lambda i: (i, 0)),
        pl.BlockSpec((LATENT, IN_DIM), lambda i: (0, 0)),
        pl.BlockSpec((1, IN_DIM), lambda i: (0, 0)),
    ],
    out_specs=[
        pl.BlockSpec((XB, IN_DIM), lambda i: (i, 0)),
        pl.BlockSpec((XB, 5, LATENT), lambda i: (i, 0, 0)),
    ],
    out_shape=[
        jax.ShapeDtypeStruct((B, IN_DIM), jnp.float32),
        jax.ShapeDtypeStruct((B, 5, LATENT), jnp.float32),
    ],
)


@functools.cache
def _make_sc_gather():
  @functools.partial(
    pl.kernel,
    mesh=plsc.VectorSubcoreMesh(core_axis_name="c", subcore_axis_name="s"),
    out_type=[
        jax.ShapeDtypeStruct((B, LATENT), jnp.float32),
        jax.ShapeDtypeStruct((B, LATENT), jnp.float32),
        jax.ShapeDtypeStruct((B, LATENT), jnp.float32),
        jax.ShapeDtypeStruct((B, LATENT), jnp.float32),
    ],
    scratch_types=[
        pltpu.VMEM((BPW,), jnp.int32),
        pltpu.VMEM((BPW,), jnp.int32),
        pltpu.VMEM((BPW,), jnp.int32),
        pltpu.VMEM((BPW,), jnp.int32),
        pltpu.VMEM((BPW, LATENT), jnp.float32),
        pltpu.SemaphoreType.DMA,
    ],
  )
  def _sc_gather(table_hbm, k_hbm, zq_hbm, zup_hbm, zdn_hbm, zlf_hbm,
                 kv, iu, idn, ilf, rows, sem):
    wid = lax.axis_index("s") * NC + lax.axis_index("c")
    base = wid * BPW
    pltpu.sync_copy(k_hbm.at[pl.ds(base, BPW)], kv)
    for c in range(BPW // 16):
        sl = pl.ds(c * 16, 16)
        kk = kv[sl]
        k1 = lax.shift_right_logical(kk, 7)
        k2 = jnp.bitwise_and(kk, SOM1 - 1)
        iu[sl] = jnp.where(k1 < SOM0 - 1, kk + SOM1, ZROW)
        idn[sl] = jnp.where(k1 > 0, kk - SOM1, ZROW)
        ilf[sl] = jnp.where(k2 > 0, kk - 1, ZROW)
    bsl = pl.ds(base, BPW)
    pltpu.async_copy(table_hbm.at[kv], rows, sem).wait()
    pltpu.sync_copy(rows, zq_hbm.at[bsl])
    pltpu.async_copy(table_hbm.at[iu], rows, sem).wait()
    pltpu.sync_copy(rows, zup_hbm.at[bsl])
    pltpu.async_copy(table_hbm.at[idn], rows, sem).wait()
    pltpu.sync_copy(rows, zdn_hbm.at[bsl])
    pltpu.async_copy(table_hbm.at[ilf], rows, sem).wait()
    pltpu.sync_copy(rows, zlf_hbm.at[bsl])

  return _sc_gather


def kernel(x, W_enc, b_enc, embeddings, W_dq, b_dq, W_de, b_de):
    emb_flat = embeddings.reshape(K, LATENT)
    z_e, z_dist, k_blk, x_e = _tc_call(
        x, W_enc, b_enc.reshape(1, LATENT), emb_flat,
        W_de, b_de.reshape(1, IN_DIM))
    k = k_blk.reshape(B)
    table = jnp.concatenate(
        [emb_flat, jnp.zeros((KPAD, LATENT), jnp.float32)], axis=0)
    z_q, z_up, z_dn, z_lf = _make_sc_gather()(table, k)
    x_q, z_q_neighbors = _xq_call(z_q, z_up, z_dn, z_lf,
                                  W_dq, b_dq.reshape(1, IN_DIM))
    return (x_e, x_q, z_e, z_q, z_q_neighbors, k, z_dist)


# speedup vs baseline: 1.1238x; 1.1238x over previous
import functools

import jax
import jax.numpy as jnp
from jax import lax
from jax.experimental import pallas as pl
from jax.experimental.pallas import tpu as pltpu
from jax.experimental.pallas import tpu_sc as plsc

SOM0, SOM1 = 64, 128
K = SOM0 * SOM1
LATENT = 256
IN_DIM = 1024
B = 4096

BB = 128
NB = B // BB
KPAD = 8
ZROW = K

NC, NS = 2, 16
NW = NC * NS
BPW = B // NW
CH = 32
NCH = BPW // CH
LV = LATENT // 16

XB = 512
NXB = B // XB


def _tc_body(x_ref, we_ref, be_ref, emb_ref, wde_ref, bde_ref,
             ze_ref, dist_ref, k_ref, xe_ref, tab_ref,
             e2_ref, semb_ref, ids_ref):
    i = pl.program_id(0)

    @pl.when(i == 0)
    def _init():
        emb = emb_ref[...]
        e2_ref[...] = jnp.sum(emb * emb, axis=1)[None, :]
        semb_ref[...] = -2.0 * emb
        ids_ref[...] = lax.broadcasted_iota(jnp.int32, (1, K), 1)
        tab_ref[pl.ds(0, K), :] = emb
        tab_ref[pl.ds(K, KPAD), :] = jnp.zeros((KPAD, LATENT), jnp.float32)

    z = jnp.dot(x_ref[...], we_ref[...],
                preferred_element_type=jnp.float32) + be_ref[...]
    ze_ref[...] = z
    z2 = jnp.sum(z * z, axis=1, keepdims=True)
    cross2 = lax.dot_general(z, semb_ref[...], (((1,), (1,)), ((), ())),
                             preferred_element_type=jnp.float32)
    dist = (z2 + cross2) + e2_ref[...]
    dist_ref[...] = dist
    m = jnp.min(dist, axis=1, keepdims=True)
    k_ref[0, 0, :] = jnp.min(jnp.where(dist == m, ids_ref[...], K), axis=1)
    xe_ref[...] = jnp.dot(z, wde_ref[...],
                          preferred_element_type=jnp.float32) + bde_ref[...]


_tc_call = pl.pallas_call(
    _tc_body,
    grid=(NB,),
    in_specs=[
        pl.BlockSpec((BB, IN_DIM), lambda i: (i, 0)),
        pl.BlockSpec((IN_DIM, LATENT), lambda i: (0, 0)),
        pl.BlockSpec((1, LATENT), lambda i: (0, 0)),
        pl.BlockSpec((K, LATENT), lambda i: (0, 0)),
        pl.BlockSpec((LATENT, IN_DIM), lambda i: (0, 0)),
        pl.BlockSpec((1, IN_DIM), lambda i: (0, 0)),
    ],
    out_specs=[
        pl.BlockSpec((BB, LATENT), lambda i: (i, 0)),
        pl.BlockSpec((BB, K), lambda i: (i, 0)),
        pl.BlockSpec((1, 1, BB), lambda i: (i, 0, 0)),
        pl.BlockSpec((BB, IN_DIM), lambda i: (i, 0)),
        pl.BlockSpec((K + KPAD, LATENT), lambda i: (0, 0)),
    ],
    out_shape=[
        jax.ShapeDtypeStruct((B, LATENT), jnp.float32),
        jax.ShapeDtypeStruct((B, K), jnp.float32),
        jax.ShapeDtypeStruct((NB, 1, BB), jnp.int32),
        jax.ShapeDtypeStruct((B, IN_DIM), jnp.float32),
        jax.ShapeDtypeStruct((K + KPAD, LATENT), jnp.float32),
    ],
    scratch_shapes=[
        pltpu.VMEM((1, K), jnp.float32),
        pltpu.VMEM((K, LATENT), jnp.float32),
        pltpu.VMEM((1, K), jnp.int32),
    ],
)


def _xq_body(zq_ref, wdq_ref, bdq_ref, xq_ref):
    xq_ref[...] = jnp.dot(zq_ref[...], wdq_ref[...],
                          preferred_element_type=jnp.float32) + bdq_ref[...]


_xq_call = pl.pallas_call(
    _xq_body,
    grid=(NXB,),
    in_specs=[
        pl.BlockSpec((XB, LATENT), lambda i: (i, 0)),
        pl.BlockSpec((LATENT, IN_DIM), lambda i: (0, 0)),
        pl.BlockSpec((1, IN_DIM), lambda i: (0, 0)),
    ],
    out_specs=[pl.BlockSpec((XB, IN_DIM), lambda i: (i, 0))],
    out_shape=[jax.ShapeDtypeStruct((B, IN_DIM), jnp.float32)],
)


@functools.cache
def _make_sc_gather():
  @functools.partial(
    pl.kernel,
    mesh=plsc.VectorSubcoreMesh(core_axis_name="c", subcore_axis_name="s"),
    out_type=[
        jax.ShapeDtypeStruct((B, LATENT), jnp.float32),
        jax.ShapeDtypeStruct((B, LATENT), jnp.float32),
        jax.ShapeDtypeStruct((B, LATENT), jnp.float32),
        jax.ShapeDtypeStruct((B, LATENT), jnp.float32),
    ],
    scratch_types=[
        pltpu.VMEM((BPW,), jnp.int32),
        pltpu.VMEM((BPW,), jnp.int32),
        pltpu.VMEM((BPW,), jnp.int32),
        pltpu.VMEM((BPW,), jnp.int32),
        pltpu.VMEM((BPW, LATENT), jnp.float32),
        pltpu.SemaphoreType.DMA,
    ],
  )
  def _sc_gather(table_hbm, k_hbm, zq_hbm, zup_hbm, zdn_hbm, zlf_hbm,
                 kv, iu, idn, ilf, rows, sem):
    wid = lax.axis_index("s") * NC + lax.axis_index("c")
    base = wid * BPW
    pltpu.sync_copy(k_hbm.at[pl.ds(base, BPW)], kv)
    for c in range(BPW // 16):
        sl = pl.ds(c * 16, 16)
        kk = kv[sl]
        k1 = lax.shift_right_logical(kk, 7)
        k2 = jnp.bitwise_and(kk, SOM1 - 1)
        iu[sl] = jnp.where(k1 < SOM0 - 1, kk + SOM1, ZROW)
        idn[sl] = jnp.where(k1 > 0, kk - SOM1, ZROW)
        ilf[sl] = jnp.where(k2 > 0, kk - 1, ZROW)
    bsl = pl.ds(base, BPW)
    pltpu.async_copy(table_hbm.at[kv], rows, sem).wait()
    pltpu.sync_copy(rows, zq_hbm.at[bsl])
    pltpu.async_copy(table_hbm.at[iu], rows, sem).wait()
    pltpu.sync_copy(rows, zup_hbm.at[bsl])
    pltpu.async_copy(table_hbm.at[idn], rows, sem).wait()
    pltpu.sync_copy(rows, zdn_hbm.at[bsl])
    pltpu.async_copy(table_hbm.at[ilf], rows, sem).wait()
    pltpu.sync_copy(rows, zlf_hbm.at[bsl])

  return _sc_gather


def kernel(x, W_enc, b_enc, embeddings, W_dq, b_dq, W_de, b_de):
    emb_flat = embeddings.reshape(K, LATENT)
    z_e, z_dist, k_blk, x_e, table = _tc_call(
        x, W_enc, b_enc.reshape(1, LATENT), emb_flat,
        W_de, b_de.reshape(1, IN_DIM))
    k = k_blk.reshape(B)
    z_q, z_up, z_dn, z_lf = _make_sc_gather()(table, k)
    z_rt = jnp.zeros((B, LATENT), jnp.float32)
    z_q_neighbors = jnp.stack([z_q, z_up, z_dn, z_rt, z_lf], axis=1)
    (x_q,) = _xq_call(z_q, W_dq, b_dq.reshape(1, IN_DIM))
    return (x_e, x_q, z_e, z_q, z_q_neighbors, k, z_dist)


# speedup vs baseline: 1.3682x; 1.2175x over previous
import functools

import jax
import jax.numpy as jnp
from jax import lax
from jax.experimental import pallas as pl
from jax.experimental.pallas import tpu as pltpu
from jax.experimental.pallas import tpu_sc as plsc

SOM0, SOM1 = 64, 128
K = SOM0 * SOM1
LATENT = 256
IN_DIM = 1024
B = 4096

BB = 256
NB = B // BB
KPAD = 8
ZROW = K

NC, NS = 2, 16
NW = NC * NS
BPW = B // NW
CH = 32
NCH = BPW // CH
LV = LATENT // 16

XB = 512
NXB = B // XB


def _tc_body(x_ref, we_ref, be_ref, emb_ref, wde_ref, bde_ref,
             ze_ref, dist_ref, k_ref, xe_ref, tab_ref,
             e2_ref, semb_ref, ids_ref):
    i = pl.program_id(0)

    @pl.when(i == 0)
    def _init():
        emb = emb_ref[...]
        e2_ref[...] = jnp.sum(emb * emb, axis=1)[None, :]
        semb_ref[...] = -2.0 * emb
        ids_ref[...] = lax.broadcasted_iota(jnp.int32, (1, K), 1)
        tab_ref[pl.ds(0, K), :] = emb
        tab_ref[pl.ds(K, KPAD), :] = jnp.zeros((KPAD, LATENT), jnp.float32)

    z = jnp.dot(x_ref[...], we_ref[...],
                preferred_element_type=jnp.float32) + be_ref[...]
    ze_ref[...] = z
    z2 = jnp.sum(z * z, axis=1, keepdims=True)
    cross2 = lax.dot_general(z, semb_ref[...], (((1,), (1,)), ((), ())),
                             preferred_element_type=jnp.float32)
    dist = (z2 + cross2) + e2_ref[...]
    dist_ref[...] = dist
    m = jnp.min(dist, axis=1, keepdims=True)
    k_ref[0, 0, :] = jnp.min(jnp.where(dist == m, ids_ref[...], K), axis=1)
    xe_ref[...] = jnp.dot(z, wde_ref[...],
                          preferred_element_type=jnp.float32) + bde_ref[...]


_tc_call = pl.pallas_call(
    _tc_body,
    grid=(NB,),
    in_specs=[
        pl.BlockSpec((BB, IN_DIM), lambda i: (i, 0)),
        pl.BlockSpec((IN_DIM, LATENT), lambda i: (0, 0)),
        pl.BlockSpec((1, LATENT), lambda i: (0, 0)),
        pl.BlockSpec((K, LATENT), lambda i: (0, 0)),
        pl.BlockSpec((LATENT, IN_DIM), lambda i: (0, 0)),
        pl.BlockSpec((1, IN_DIM), lambda i: (0, 0)),
    ],
    out_specs=[
        pl.BlockSpec((BB, LATENT), lambda i: (i, 0)),
        pl.BlockSpec((BB, K), lambda i: (i, 0)),
        pl.BlockSpec((1, 1, BB), lambda i: (i, 0, 0)),
        pl.BlockSpec((BB, IN_DIM), lambda i: (i, 0)),
        pl.BlockSpec((K + KPAD, LATENT), lambda i: (0, 0)),
    ],
    out_shape=[
        jax.ShapeDtypeStruct((B, LATENT), jnp.float32),
        jax.ShapeDtypeStruct((B, K), jnp.float32),
        jax.ShapeDtypeStruct((NB, 1, BB), jnp.int32),
        jax.ShapeDtypeStruct((B, IN_DIM), jnp.float32),
        jax.ShapeDtypeStruct((K + KPAD, LATENT), jnp.float32),
    ],
    scratch_shapes=[
        pltpu.VMEM((1, K), jnp.float32),
        pltpu.VMEM((K, LATENT), jnp.float32),
        pltpu.VMEM((1, K), jnp.int32),
    ],
)


def _xq_body(zq_ref, wdq_ref, bdq_ref, xq_ref):
    xq_ref[...] = jnp.dot(zq_ref[...], wdq_ref[...],
                          preferred_element_type=jnp.float32) + bdq_ref[...]


_xq_call = pl.pallas_call(
    _xq_body,
    grid=(NXB,),
    in_specs=[
        pl.BlockSpec((XB, LATENT), lambda i: (i, 0)),
        pl.BlockSpec((LATENT, IN_DIM), lambda i: (0, 0)),
        pl.BlockSpec((1, IN_DIM), lambda i: (0, 0)),
    ],
    out_specs=[pl.BlockSpec((XB, IN_DIM), lambda i: (i, 0))],
    out_shape=[jax.ShapeDtypeStruct((B, IN_DIM), jnp.float32)],
)


@functools.cache
def _make_sc_gather():
  @functools.partial(
    pl.kernel,
    mesh=plsc.VectorSubcoreMesh(core_axis_name="c", subcore_axis_name="s"),
    out_type=[
        jax.ShapeDtypeStruct((B, LATENT), jnp.float32),
        jax.ShapeDtypeStruct((B, LATENT), jnp.float32),
        jax.ShapeDtypeStruct((B, LATENT), jnp.float32),
        jax.ShapeDtypeStruct((B, LATENT), jnp.float32),
    ],
    scratch_types=[
        pltpu.VMEM((BPW,), jnp.int32),
        pltpu.VMEM((BPW,), jnp.int32),
        pltpu.VMEM((BPW,), jnp.int32),
        pltpu.VMEM((BPW,), jnp.int32),
        pltpu.VMEM((BPW, LATENT), jnp.float32),
        pltpu.SemaphoreType.DMA,
    ],
  )
  def _sc_gather(table_hbm, k_hbm, zq_hbm, zup_hbm, zdn_hbm, zlf_hbm,
                 kv, iu, idn, ilf, rows, sem):
    wid = lax.axis_index("s") * NC + lax.axis_index("c")
    base = wid * BPW
    pltpu.sync_copy(k_hbm.at[pl.ds(base, BPW)], kv)
    for c in range(BPW // 16):
        sl = pl.ds(c * 16, 16)
        kk = kv[sl]
        k1 = lax.shift_right_logical(kk, 7)
        k2 = jnp.bitwise_and(kk, SOM1 - 1)
        iu[sl] = jnp.where(k1 < SOM0 - 1, kk + SOM1, ZROW)
        idn[sl] = jnp.where(k1 > 0, kk - SOM1, ZROW)
        ilf[sl] = jnp.where(k2 > 0, kk - 1, ZROW)
    bsl = pl.ds(base, BPW)
    pltpu.async_copy(table_hbm.at[kv], rows, sem).wait()
    pltpu.sync_copy(rows, zq_hbm.at[bsl])
    pltpu.async_copy(table_hbm.at[iu], rows, sem).wait()
    pltpu.sync_copy(rows, zup_hbm.at[bsl])
    pltpu.async_copy(table_hbm.at[idn], rows, sem).wait()
    pltpu.sync_copy(rows, zdn_hbm.at[bsl])
    pltpu.async_copy(table_hbm.at[ilf], rows, sem).wait()
    pltpu.sync_copy(rows, zlf_hbm.at[bsl])

  return _sc_gather


def kernel(x, W_enc, b_enc, embeddings, W_dq, b_dq, W_de, b_de):
    emb_flat = embeddings.reshape(K, LATENT)
    z_e, z_dist, k_blk, x_e, table = _tc_call(
        x, W_enc, b_enc.reshape(1, LATENT), emb_flat,
        W_de, b_de.reshape(1, IN_DIM))
    k = k_blk.reshape(B)
    z_q, z_up, z_dn, z_lf = _make_sc_gather()(table, k)
    z_rt = jnp.zeros((B, LATENT), jnp.float32)
    z_q_neighbors = jnp.stack([z_q, z_up, z_dn, z_rt, z_lf], axis=1)
    (x_q,) = _xq_call(z_q, W_dq, b_dq.reshape(1, IN_DIM))
    return (x_e, x_q, z_e, z_q, z_q_neighbors, k, z_dist)
